# baseline (speedup 1.0000x reference)
import functools

import jax
import jax.numpy as jnp
from jax import lax
from jax.experimental import pallas as pl
from jax.experimental.pallas import tpu as pltpu
from jax.experimental.pallas import tpu_sc as plsc

NUM_MAPS = 4
KMAX = 20
KMIN = 20
ALPHA = 0.7

L = 16
NC = 2
NS = 16
NW = NC * NS

B = 64
CH = 768
HW = 24
N = NUM_MAPS * HW * HW
ROWS = B * (CH // NUM_MAPS)
ROWS_PER_W = ROWS // NW

RPC = 4
CPC = RPC * NUM_MAPS
NCH = ROWS_PER_W // RPC
CH_PER_BATCH = NCH // 2

IL = RPC
NB = 4
CAP = 2368
CAPH = CAP // 2
SMERGE = 3

NEG = -3.0e38


def _sort_desc(v):
  k, _ = plsc.sort_key_val(v, v, descending=True)
  return k


def _merge_halves(a_desc, b_desc):
  b_asc = lax.rev(b_desc, (0,))
  return jnp.maximum(a_desc, b_asc), jnp.minimum(a_desc, b_asc)


def _merge32(b1, b2, s_desc):
  t_hi, _ = _merge_halves(b2, s_desc)
  t_hi = _sort_desc(t_hi)
  u, w = _merge_halves(b1, t_hi)
  return _sort_desc(u), _sort_desc(w)


def _sc_body(x_hbm, out_hbm, bufs, cand, obuf, sem0, sem1):
  wid = lax.axis_index("s") * NC + lax.axis_index("c")
  lane = lax.iota(jnp.int32, L)
  upper = lane >= 8

  ch_base = wid * (ROWS_PER_W * NUM_MAPS)

  def start_chunk(c, sel_static, sem):
    pltpu.async_copy(
        x_hbm.at[pl.ds(ch_base + c * CPC, CPC)], bufs.at[sel_static], sem)

  def wait_chunk(sel_static, sem):
    pltpu.make_async_copy(
        x_hbm.at[pl.ds(0, CPC)], bufs.at[sel_static], sem).wait()

  def loads(sel, ch, h):
    a = bufs[sel, ch, h, pl.ds(0, L)]
    b = bufs[sel, ch, h, pl.ds(8, L)]
    return a, b

  def fold_row(sel, ch0):
    def fold_body(i, carry):
      amax = list(carry[:NB])
      amin = list(carry[NB:])
      for k in range(NUM_MAPS):
        a, b = loads(sel, ch0 + k, i)
        bmax = jnp.where(upper, b, NEG)
        bmin = jnp.where(upper, b, -NEG)
        amax[2 * (k % 2)] = jnp.maximum(amax[2 * (k % 2)], a)
        amax[2 * (k % 2) + 1] = jnp.maximum(amax[2 * (k % 2) + 1], bmax)
        amin[2 * (k % 2)] = jnp.minimum(amin[2 * (k % 2)], a)
        amin[2 * (k % 2) + 1] = jnp.minimum(amin[2 * (k % 2) + 1], bmin)
      return tuple(amax) + tuple(amin)

    init = tuple([jnp.full((L,), NEG, jnp.float32)] * NB) + tuple(
        [jnp.full((L,), -NEG, jnp.float32)] * NB)
    accs = lax.fori_loop(0, HW, fold_body, init)
    return accs[:NB], accs[NB:]

  def nth20(vregs):
    s = [_sort_desc(v) for v in vregs]
    hi, lo = _merge_halves(s[0], s[1])
    b1, b2 = _sort_desc(hi), _sort_desc(lo)
    for k in range(2, NB):
      b1, b2 = _merge32(b1, b2, s[k])
    return jnp.max(jnp.where(lane == (KMAX - L - 1), b2, NEG))

  def process_chunk(sel, lane0):
    th_splat = []
    tl_splat = []
    for r in range(IL):
      amax, amin = fold_row(sel, NUM_MAPS * r)
      t_hi = nth20(amax)
      t_lo = -nth20([-v for v in amin])
      th_splat.append(jnp.full((L,), t_hi, jnp.float32))
      tl_splat.append(jnp.full((L,), t_lo, jnp.float32))

    def filt_body(i, carry):
      cnt = list(carry)
      for r in range(IL):
        for half in range(2):
          seg = r * CAP + half * CAPH
          ci = 2 * r + half
          for k in range(NUM_MAPS):
            a, b = loads(sel, NUM_MAPS * r + k, 2 * i + half)
            ma = (a >= th_splat[r]) | (a <= tl_splat[r])
            mb = ((b >= th_splat[r]) | (b <= tl_splat[r])) & upper
            plsc.store_compressed(
                cand.at[pl.ds(seg + cnt[ci], L)], a, mask=ma)
            c2 = cnt[ci] + jnp.sum(ma.astype(jnp.int32))
            plsc.store_compressed(cand.at[pl.ds(seg + c2, L)], b, mask=mb)
            cnt[ci] = c2 + jnp.sum(mb.astype(jnp.int32))
      return tuple(cnt)

    cnts = lax.fori_loop(0, HW // 2, filt_body, (jnp.int32(0),) * (2 * IL))

    def masked_cand(r, half, i, negate):
      v = cand[pl.ds(r * CAP + half * CAPH + i * L, L)]
      if negate:
        v = -v
      return jnp.where(lane < cnts[2 * r + half] - i * L, v, NEG)

    def static_merges(negate):
      b1 = [jnp.full((L,), NEG, jnp.float32) for _ in range(IL)]
      b2 = [jnp.full((L,), NEG, jnp.float32) for _ in range(IL)]
      for i in range(SMERGE):
        for r in range(IL):
          for half in range(2):
            b1[r], b2[r] = _merge32(
                b1[r], b2[r], _sort_desc(masked_cand(r, half, i, negate)))
      return b1, b2

    def dyn_tail(b1, b2, r, negate):
      for half in range(2):
        nv = lax.shift_right_logical(cnts[2 * r + half] + (L - 1), 4)

        def mbody(i, carry):
          return _merge32(*carry, _sort_desc(masked_cand(r, half, i, negate)))

        b1, b2 = lax.fori_loop(SMERGE, nv, mbody, (b1, b2))
      return b1, b2

    def sum20(b1, b2):
      return jnp.sum(b1) + jnp.sum(
          jnp.where(lane < KMAX - L, b2, jnp.float32(0.0)))

    h1, h2 = static_merges(False)
    l1, l2 = static_merges(True)
    acc = jnp.zeros((L,), jnp.float32)
    for r in range(IL):
      hb1, hb2 = dyn_tail(h1[r], h2[r], r, False)
      lb1, lb2 = dyn_tail(l1[r], l2[r], r, True)
      s_top = sum20(hb1, hb2)
      s_bot = -sum20(lb1, lb2)
      res = (s_top * (1.0 / KMAX) + s_bot * (ALPHA / KMIN)) * jnp.float32(0.5)
      acc = jnp.where(lane == lane0 + r, res, acc)
    return acc

  start_chunk(0, 0, sem0)
  start_chunk(1, 1, sem1)

  def chunk_body(c, acc):
    sel = jnp.bitwise_and(c, 1)

    @pl.when(sel == 0)
    def _():
      wait_chunk(0, sem0)

    @pl.when(sel == 1)
    def _():
      wait_chunk(1, sem1)

    quad = jnp.bitwise_and(c, 3)
    acc = acc + process_chunk(sel, quad * RPC)

    @pl.when(quad == 3)
    def _():
      obuf[pl.ds(lax.shift_left(lax.shift_right_logical(c, 2), 4), L)] = acc

    acc = jnp.where(quad == 3, jnp.zeros((L,), jnp.float32), acc)

    @pl.when(c + 2 < NCH)
    def _():
      @pl.when(sel == 0)
      def _():
        start_chunk(c + 2, 0, sem0)

      @pl.when(sel == 1)
      def _():
        start_chunk(c + 2, 1, sem1)

    return acc

  lax.fori_loop(0, NCH, chunk_body, jnp.zeros((L,), jnp.float32))

  pltpu.sync_copy(obuf, out_hbm.at[pl.ds(wid * ROWS_PER_W, ROWS_PER_W)])


@jax.jit
def kernel(input):
  batch, ch, h, w = input.shape
  num_outputs = ch // NUM_MAPS

  mesh = plsc.VectorSubcoreMesh(
      core_axis_name="c", subcore_axis_name="s",
      num_cores=NC, num_subcores=NS)
  run = functools.partial(
      pl.kernel,
      out_type=jax.ShapeDtypeStruct((ROWS,), jnp.float32),
      mesh=mesh,
      scratch_types=[
          pltpu.VMEM((2, CPC, HW, HW), jnp.float32),
          pltpu.VMEM((IL * CAP,), jnp.float32),
          pltpu.VMEM((ROWS_PER_W,), jnp.float32),
          pltpu.SemaphoreType.DMA,
          pltpu.SemaphoreType.DMA,
      ],
      compiler_params=pltpu.CompilerParams(
          needs_layout_passes=False, use_tc_tiling_on_sc=True),
  )(_sc_body)
  out = run(input.reshape(batch * ch, h, w))
  return out.reshape(batch, num_outputs)


# speedup vs baseline: 1.0715x; 1.0715x over previous
import functools

import jax
import jax.numpy as jnp
from jax import lax
from jax.experimental import pallas as pl
from jax.experimental.pallas import tpu as pltpu
from jax.experimental.pallas import tpu_sc as plsc

NUM_MAPS = 4
KMAX = 20
KMIN = 20
ALPHA = 0.7

L = 16
NC = 2
NS = 16
NW = NC * NS

B = 64
CH = 768
HW = 24
N = NUM_MAPS * HW * HW
ROWS = B * (CH // NUM_MAPS)
ROWS_PER_W = ROWS // NW

RPC = 4
CPC = RPC * NUM_MAPS
NCH = ROWS_PER_W // RPC
CH_PER_BATCH = NCH // 2

IL = RPC
NB = 4
CAP = N + L
SMERGE = 3

NEG = -3.0e38


def _sort_desc(v):
  k, _ = plsc.sort_key_val(v, v, descending=True)
  return k


def _merge_halves(a_desc, b_desc):
  b_asc = lax.rev(b_desc, (0,))
  return jnp.maximum(a_desc, b_asc), jnp.minimum(a_desc, b_asc)


def _merge32(b1, b2, s_desc):
  t_hi, _ = _merge_halves(b2, s_desc)
  t_hi = _sort_desc(t_hi)
  u, w = _merge_halves(b1, t_hi)
  return _sort_desc(u), _sort_desc(w)


def _sc_body(x_hbm, out_hbm, bufs, cand, obuf, sem0, sem1):
  wid = lax.axis_index("s") * NC + lax.axis_index("c")
  lane = lax.iota(jnp.int32, L)
  upper = lane >= 8

  ch_base = wid * (ROWS_PER_W * NUM_MAPS)

  def start_chunk(c, sel_static, sem):
    pltpu.async_copy(
        x_hbm.at[pl.ds(ch_base + c * CPC, CPC)], bufs.at[sel_static], sem)

  def wait_chunk(sel_static, sem):
    pltpu.make_async_copy(
        x_hbm.at[pl.ds(0, CPC)], bufs.at[sel_static], sem).wait()

  def loads(sel, ch, h):
    a = bufs[sel, ch, h, pl.ds(0, L)]
    b = bufs[sel, ch, h, pl.ds(8, L)]
    return a, b

  def fold_row(sel, ch0):
    def fold_body(i, carry):
      amax = list(carry[:NB])
      amin = list(carry[NB:])
      for k in range(NUM_MAPS):
        a, b = loads(sel, ch0 + k, i)
        bmax = jnp.where(upper, b, NEG)
        bmin = jnp.where(upper, b, -NEG)
        amax[2 * (k % 2)] = jnp.maximum(amax[2 * (k % 2)], a)
        amax[2 * (k % 2) + 1] = jnp.maximum(amax[2 * (k % 2) + 1], bmax)
        amin[2 * (k % 2)] = jnp.minimum(amin[2 * (k % 2)], a)
        amin[2 * (k % 2) + 1] = jnp.minimum(amin[2 * (k % 2) + 1], bmin)
      return tuple(amax) + tuple(amin)

    init = tuple([jnp.full((L,), NEG, jnp.float32)] * NB) + tuple(
        [jnp.full((L,), -NEG, jnp.float32)] * NB)
    accs = lax.fori_loop(0, HW, fold_body, init)
    return accs[:NB], accs[NB:]

  def nth20(vregs):
    s = [_sort_desc(v) for v in vregs]
    hi, lo = _merge_halves(s[0], s[1])
    b1, b2 = _sort_desc(hi), _sort_desc(lo)
    for k in range(2, NB):
      b1, b2 = _merge32(b1, b2, s[k])
    return jnp.max(jnp.where(lane == (KMAX - L - 1), b2, NEG))

  def process_chunk(sel, lane0):
    th_splat = []
    tl_splat = []
    for r in range(IL):
      amax, amin = fold_row(sel, NUM_MAPS * r)
      t_hi = nth20(amax)
      t_lo = -nth20([-v for v in amin])
      th_splat.append(jnp.full((L,), t_hi, jnp.float32))
      tl_splat.append(jnp.full((L,), t_lo, jnp.float32))

    def filt_body(i, carry):
      cnt = list(carry)
      for r in range(IL):
        for k in range(NUM_MAPS):
          a, b = loads(sel, NUM_MAPS * r + k, i)
          ma = (a >= th_splat[r]) | (a <= tl_splat[r])
          mb = ((b >= th_splat[r]) | (b <= tl_splat[r])) & upper
          plsc.store_compressed(
              cand.at[pl.ds(r * CAP + cnt[r], L)], a, mask=ma)
          c2 = cnt[r] + jnp.sum(ma.astype(jnp.int32))
          plsc.store_compressed(cand.at[pl.ds(r * CAP + c2, L)], b, mask=mb)
          cnt[r] = c2 + jnp.sum(mb.astype(jnp.int32))
      return tuple(cnt)

    cnts = lax.fori_loop(0, HW, filt_body, (jnp.int32(0),) * IL)

    def masked_cand(r, i, negate):
      v = cand[pl.ds(r * CAP + i * L, L)]
      if negate:
        v = -v
      return jnp.where(lane < cnts[r] - i * L, v, NEG)

    def static_merges(negate):
      b1 = [jnp.full((L,), NEG, jnp.float32) for _ in range(IL)]
      b2 = [jnp.full((L,), NEG, jnp.float32) for _ in range(IL)]
      for i in range(SMERGE):
        for r in range(IL):
          b1[r], b2[r] = _merge32(
              b1[r], b2[r], _sort_desc(masked_cand(r, i, negate)))
      return b1, b2

    def dyn_tail(b1, b2, r, negate):
      nv = lax.shift_right_logical(cnts[r] + (L - 1), 4)

      def mbody(i, carry):
        return _merge32(*carry, _sort_desc(masked_cand(r, i, negate)))

      return lax.fori_loop(SMERGE, nv, mbody, (b1, b2))

    def sum20(b1, b2):
      return jnp.sum(b1) + jnp.sum(
          jnp.where(lane < KMAX - L, b2, jnp.float32(0.0)))

    h1, h2 = static_merges(False)
    l1, l2 = static_merges(True)
    acc = jnp.zeros((L,), jnp.float32)
    for r in range(IL):
      hb1, hb2 = dyn_tail(h1[r], h2[r], r, False)
      lb1, lb2 = dyn_tail(l1[r], l2[r], r, True)
      s_top = sum20(hb1, hb2)
      s_bot = -sum20(lb1, lb2)
      res = (s_top * (1.0 / KMAX) + s_bot * (ALPHA / KMIN)) * jnp.float32(0.5)
      acc = jnp.where(lane == lane0 + r, res, acc)
    return acc

  start_chunk(0, 0, sem0)
  start_chunk(1, 1, sem1)

  def chunk_body(c, acc):
    sel = jnp.bitwise_and(c, 1)

    @pl.when(sel == 0)
    def _():
      wait_chunk(0, sem0)

    @pl.when(sel == 1)
    def _():
      wait_chunk(1, sem1)

    quad = jnp.bitwise_and(c, 3)
    acc = acc + process_chunk(sel, quad * RPC)

    @pl.when(quad == 3)
    def _():
      obuf[pl.ds(lax.shift_left(lax.shift_right_logical(c, 2), 4), L)] = acc

    acc = jnp.where(quad == 3, jnp.zeros((L,), jnp.float32), acc)

    @pl.when(c + 2 < NCH)
    def _():
      @pl.when(sel == 0)
      def _():
        start_chunk(c + 2, 0, sem0)

      @pl.when(sel == 1)
      def _():
        start_chunk(c + 2, 1, sem1)

    return acc

  lax.fori_loop(0, NCH, chunk_body, jnp.zeros((L,), jnp.float32))

  pltpu.sync_copy(obuf, out_hbm.at[pl.ds(wid * ROWS_PER_W, ROWS_PER_W)])


@jax.jit
def kernel(input):
  batch, ch, h, w = input.shape
  num_outputs = ch // NUM_MAPS

  mesh = plsc.VectorSubcoreMesh(
      core_axis_name="c", subcore_axis_name="s",
      num_cores=NC, num_subcores=NS)
  run = functools.partial(
      pl.kernel,
      out_type=jax.ShapeDtypeStruct((ROWS,), jnp.float32),
      mesh=mesh,
      scratch_types=[
          pltpu.VMEM((2, CPC, HW, HW), jnp.float32),
          pltpu.VMEM((IL * CAP,), jnp.float32),
          pltpu.VMEM((ROWS_PER_W,), jnp.float32),
          pltpu.SemaphoreType.DMA,
          pltpu.SemaphoreType.DMA,
      ],
      compiler_params=pltpu.CompilerParams(
          needs_layout_passes=False, use_tc_tiling_on_sc=True),
  )(_sc_body)
  out = run(input.reshape(batch * ch, h, w))
  return out.reshape(batch, num_outputs)


# speedup vs baseline: 1.0813x; 1.0092x over previous
import functools

import jax
import jax.numpy as jnp
from jax import lax
from jax.experimental import pallas as pl
from jax.experimental.pallas import tpu as pltpu
from jax.experimental.pallas import tpu_sc as plsc

NUM_MAPS = 4
KMAX = 20
KMIN = 20
ALPHA = 0.7

L = 16
NC = 2
NS = 16
NW = NC * NS

B = 64
CH = 768
HW = 24
N = NUM_MAPS * HW * HW
ROWS = B * (CH // NUM_MAPS)
ROWS_PER_W = ROWS // NW

RPC = 4
CPC = RPC * NUM_MAPS
NCH = ROWS_PER_W // RPC
CH_PER_BATCH = NCH // 2

IL = RPC
NB = 4
CAP = N + L
SMERGE = 4

NEG = -3.0e38


def _sort_desc(v):
  k, _ = plsc.sort_key_val(v, v, descending=True)
  return k


def _merge_halves(a_desc, b_desc):
  b_asc = lax.rev(b_desc, (0,))
  return jnp.maximum(a_desc, b_asc), jnp.minimum(a_desc, b_asc)


def _merge32(b1, b2, s_desc):
  t_hi, _ = _merge_halves(b2, s_desc)
  t_hi = _sort_desc(t_hi)
  u, w = _merge_halves(b1, t_hi)
  return _sort_desc(u), _sort_desc(w)


def _sc_body(x_hbm, out_hbm, bufs, cand, obuf, sem0, sem1):
  wid = lax.axis_index("s") * NC + lax.axis_index("c")
  lane = lax.iota(jnp.int32, L)
  upper = lane >= 8

  ch_base = wid * (ROWS_PER_W * NUM_MAPS)

  def start_chunk(c, sel_static, sem):
    pltpu.async_copy(
        x_hbm.at[pl.ds(ch_base + c * CPC, CPC)], bufs.at[sel_static], sem)

  def wait_chunk(sel_static, sem):
    pltpu.make_async_copy(
        x_hbm.at[pl.ds(0, CPC)], bufs.at[sel_static], sem).wait()

  def loads(sel, ch, h):
    a = bufs[sel, ch, h, pl.ds(0, L)]
    b = bufs[sel, ch, h, pl.ds(8, L)]
    return a, b

  def fold_row(sel, ch0):
    def fold_body(i, carry):
      amax = list(carry[:NB])
      amin = list(carry[NB:])
      for k in range(NUM_MAPS):
        a, b = loads(sel, ch0 + k, i)
        bmax = jnp.where(upper, b, NEG)
        bmin = jnp.where(upper, b, -NEG)
        amax[2 * (k % 2)] = jnp.maximum(amax[2 * (k % 2)], a)
        amax[2 * (k % 2) + 1] = jnp.maximum(amax[2 * (k % 2) + 1], bmax)
        amin[2 * (k % 2)] = jnp.minimum(amin[2 * (k % 2)], a)
        amin[2 * (k % 2) + 1] = jnp.minimum(amin[2 * (k % 2) + 1], bmin)
      return tuple(amax) + tuple(amin)

    init = tuple([jnp.full((L,), NEG, jnp.float32)] * NB) + tuple(
        [jnp.full((L,), -NEG, jnp.float32)] * NB)
    accs = lax.fori_loop(0, HW, fold_body, init)
    return accs[:NB], accs[NB:]

  def nth20(vregs):
    s = [_sort_desc(v) for v in vregs]
    hi, lo = _merge_halves(s[0], s[1])
    b1, b2 = _sort_desc(hi), _sort_desc(lo)
    for k in range(2, NB):
      b1, b2 = _merge32(b1, b2, s[k])
    return jnp.max(jnp.where(lane == (KMAX - L - 1), b2, NEG))

  def process_chunk(sel, lane0):
    th_splat = []
    tl_splat = []
    for r in range(IL):
      amax, amin = fold_row(sel, NUM_MAPS * r)
      t_hi = nth20(amax)
      t_lo = -nth20([-v for v in amin])
      th_splat.append(jnp.full((L,), t_hi, jnp.float32))
      tl_splat.append(jnp.full((L,), t_lo, jnp.float32))

    def filt_body(i, carry):
      cnt = list(carry)
      for r in range(IL):
        for k in range(NUM_MAPS):
          a, b = loads(sel, NUM_MAPS * r + k, i)
          ma = (a >= th_splat[r]) | (a <= tl_splat[r])
          mb = ((b >= th_splat[r]) | (b <= tl_splat[r])) & upper
          plsc.store_compressed(
              cand.at[pl.ds(r * CAP + cnt[r], L)], a, mask=ma)
          c2 = cnt[r] + jnp.sum(ma.astype(jnp.int32))
          plsc.store_compressed(cand.at[pl.ds(r * CAP + c2, L)], b, mask=mb)
          cnt[r] = c2 + jnp.sum(mb.astype(jnp.int32))
      return tuple(cnt)

    cnts = lax.fori_loop(0, HW, filt_body, (jnp.int32(0),) * IL)

    def masked_cand(r, i, negate):
      v = cand[pl.ds(r * CAP + i * L, L)]
      if negate:
        v = -v
      return jnp.where(lane < cnts[r] - i * L, v, NEG)

    def static_merges(negate):
      b1 = [jnp.full((L,), NEG, jnp.float32) for _ in range(IL)]
      b2 = [jnp.full((L,), NEG, jnp.float32) for _ in range(IL)]
      for i in range(SMERGE):
        for r in range(IL):
          b1[r], b2[r] = _merge32(
              b1[r], b2[r], _sort_desc(masked_cand(r, i, negate)))
      return b1, b2

    def dyn_tail(b1, b2, r, negate):
      nv = lax.shift_right_logical(cnts[r] + (L - 1), 4)

      def mbody(i, carry):
        return _merge32(*carry, _sort_desc(masked_cand(r, i, negate)))

      return lax.fori_loop(SMERGE, nv, mbody, (b1, b2))

    def sum20(b1, b2):
      return jnp.sum(b1) + jnp.sum(
          jnp.where(lane < KMAX - L, b2, jnp.float32(0.0)))

    h1, h2 = static_merges(False)
    l1, l2 = static_merges(True)
    acc = jnp.zeros((L,), jnp.float32)
    for r in range(IL):
      hb1, hb2 = dyn_tail(h1[r], h2[r], r, False)
      lb1, lb2 = dyn_tail(l1[r], l2[r], r, True)
      s_top = sum20(hb1, hb2)
      s_bot = -sum20(lb1, lb2)
      res = (s_top * (1.0 / KMAX) + s_bot * (ALPHA / KMIN)) * jnp.float32(0.5)
      acc = jnp.where(lane == lane0 + r, res, acc)
    return acc

  start_chunk(0, 0, sem0)
  start_chunk(1, 1, sem1)

  def chunk_body(c, acc):
    sel = jnp.bitwise_and(c, 1)

    @pl.when(sel == 0)
    def _():
      wait_chunk(0, sem0)

    @pl.when(sel == 1)
    def _():
      wait_chunk(1, sem1)

    quad = jnp.bitwise_and(c, 3)
    acc = acc + process_chunk(sel, quad * RPC)

    @pl.when(quad == 3)
    def _():
      obuf[pl.ds(lax.shift_left(lax.shift_right_logical(c, 2), 4), L)] = acc

    acc = jnp.where(quad == 3, jnp.zeros((L,), jnp.float32), acc)

    @pl.when(c + 2 < NCH)
    def _():
      @pl.when(sel == 0)
      def _():
        start_chunk(c + 2, 0, sem0)

      @pl.when(sel == 1)
      def _():
        start_chunk(c + 2, 1, sem1)

    return acc

  lax.fori_loop(0, NCH, chunk_body, jnp.zeros((L,), jnp.float32))

  pltpu.sync_copy(obuf, out_hbm.at[pl.ds(wid * ROWS_PER_W, ROWS_PER_W)])


@jax.jit
def kernel(input):
  batch, ch, h, w = input.shape
  num_outputs = ch // NUM_MAPS

  mesh = plsc.VectorSubcoreMesh(
      core_axis_name="c", subcore_axis_name="s",
      num_cores=NC, num_subcores=NS)
  run = functools.partial(
      pl.kernel,
      out_type=jax.ShapeDtypeStruct((ROWS,), jnp.float32),
      mesh=mesh,
      scratch_types=[
          pltpu.VMEM((2, CPC, HW, HW), jnp.float32),
          pltpu.VMEM((IL * CAP,), jnp.float32),
          pltpu.VMEM((ROWS_PER_W,), jnp.float32),
          pltpu.SemaphoreType.DMA,
          pltpu.SemaphoreType.DMA,
      ],
      compiler_params=pltpu.CompilerParams(
          needs_layout_passes=False, use_tc_tiling_on_sc=True),
  )(_sc_body)
  out = run(input.reshape(batch * ch, h, w))
  return out.reshape(batch, num_outputs)
